# Initial kernel scaffold; baseline (speedup 1.0000x reference)
import jax
import jax.numpy as jnp
from jax.experimental import pallas as pl


def kernel(x, edge_index, att, Wz, bz, Wr, br, Wh, bh, Wlz, blz, Wlr, blr, Wlh, blh, Wout, bout):
    raise NotImplementedError("write your pallas kernel here")


# speedup vs baseline: 675.6037x; 675.6037x over previous
import functools

import jax
import jax.numpy as jnp
from jax import lax
from jax.experimental import pallas as pl
from jax.experimental.pallas import tpu as pltpu
from jax.experimental.pallas import tpu_sc as plsc

N = 50000
E = 1600000
P = 12
OUT = 4
F_OUT = 10

LANES = 16
NC = 2
NS = 16
NW = NC * NS
COLS = 16
NPAD = 51200
EPW = E // NW
CH = 2000
NCH = EPW // CH
RPT = NPAD // NS
BN = 3200

_mesh = plsc.VectorSubcoreMesh(core_axis_name="c", subcore_axis_name="s",
                               num_cores=NC, num_subcores=NS)


@functools.partial(
    pl.kernel,
    out_type=jax.ShapeDtypeStruct((NW, NPAD), jnp.float32),
    mesh=_mesh,
    scratch_types=[
        pltpu.VMEM((CH,), jnp.int32),
        pltpu.VMEM((NPAD,), jnp.float32),
    ],
    compiler_params=pltpu.CompilerParams(needs_layout_passes=False,
                                         use_tc_tiling_on_sc=False),
)
def _sc_degree(col_hbm, degp_hbm, colbuf, deg):
    cid = lax.axis_index("c")
    sid = lax.axis_index("s")
    wid = sid * NC + cid
    zero = jnp.zeros((LANES,), jnp.float32)

    def zb(i, c):
        deg[pl.ds(i * LANES, LANES)] = zero
        return c

    lax.fori_loop(0, NPAD // LANES, zb, 0)

    ones = jnp.ones((LANES,), jnp.float32)

    def chunk(ci, c):
        pltpu.sync_copy(col_hbm.at[pl.ds(wid * EPW + ci * CH, CH)], colbuf)

        def inner(i, c2):
            idx = colbuf[pl.ds(i * LANES, LANES)]
            plsc.addupdate_scatter(deg, [idx], ones)
            return c2

        return lax.fori_loop(0, CH // LANES, inner, c)

    lax.fori_loop(0, NCH, chunk, 0)
    pltpu.sync_copy(deg, degp_hbm.at[wid])


@functools.partial(
    pl.kernel,
    out_type=jax.ShapeDtypeStruct((NC, NPAD, COLS), jnp.float32),
    mesh=_mesh,
    scratch_types=[
        pltpu.VMEM((CH,), jnp.int32),
        pltpu.VMEM((CH,), jnp.int32),
        pltpu.VMEM((CH, COLS), jnp.float32),
        pltpu.VMEM_SHARED((NPAD, COLS), jnp.float32),
        pltpu.SemaphoreType.DMA,
    ],
    compiler_params=pltpu.CompilerParams(needs_layout_passes=False,
                                         use_tc_tiling_on_sc=False),
)
def _sc_spmm(row_hbm, col_hbm, y_hbm, tp_hbm, rowbuf, colbuf, rows, t_sh, sem):
    cid = lax.axis_index("c")
    sid = lax.axis_index("s")
    wid = sid * NC + cid
    zero = jnp.zeros((LANES,), jnp.float32)

    def zb(i, c):
        rows[i] = zero
        return c

    lax.fori_loop(0, CH, zb, 0)
    pltpu.sync_copy(rows.at[pl.ds(0, CH)], t_sh.at[pl.ds(sid * RPT, CH)])
    pltpu.sync_copy(rows.at[pl.ds(0, RPT - CH)],
                    t_sh.at[pl.ds(sid * RPT + CH, RPT - CH)])
    plsc.subcore_barrier()

    def chunk(ci, c):
        base = wid * EPW + ci * CH
        pltpu.sync_copy(row_hbm.at[pl.ds(base, CH)], rowbuf)
        pltpu.sync_copy(col_hbm.at[pl.ds(base, CH)], colbuf)
        pltpu.async_copy(y_hbm.at[rowbuf], rows, sem).wait()
        pltpu.sync_copy(rows, t_sh.at[colbuf], add=True)
        return c

    lax.fori_loop(0, NCH, chunk, 0)
    plsc.subcore_barrier()
    pltpu.sync_copy(t_sh.at[pl.ds(sid * RPT, RPT)],
                    tp_hbm.at[cid, pl.ds(sid * RPT, RPT)])


def _tc_prep_body(degp_ref, x16_ref, dinv_ref, y_ref):
    ones = jnp.ones((NW, 1), jnp.float32)
    deg = lax.dot_general(degp_ref[...], ones, (((0,), (0,)), ((), ())),
                          precision=lax.Precision.HIGHEST,
                          preferred_element_type=jnp.float32)
    dinv = lax.rsqrt(deg + 1.0)
    dinv_ref[...] = dinv
    y_ref[...] = dinv * x16_ref[...]


def _tc_final_body(tp_ref, y_ref, dinv_ref, exp_ref, uz_ref, cz_ref,
                   uh_ref, ch_ref, wb_ref, bo_ref, out_ref):
    t = tp_ref[0] + tp_ref[1]
    s16 = dinv_ref[...] * (t + y_ref[...])
    se = jnp.dot(s16, exp_ref[...], precision=lax.Precision.HIGHEST,
                 preferred_element_type=jnp.float32)
    z = 1.0 / (1.0 + jnp.exp(-(se * uz_ref[...] + cz_ref[...])))
    ht = jnp.tanh(se * uh_ref[...] + ch_ref[...])
    m = (1.0 - z) * ht
    out_ref[...] = (jnp.dot(m, wb_ref[...], precision=lax.Precision.HIGHEST,
                            preferred_element_type=jnp.float32)
                    + bo_ref[...])


def kernel(x, edge_index, att, Wz, bz, Wr, br, Wh, bh,
           Wlz, blz, Wlr, blr, Wlh, blh, Wout, bout):
    xf = x[:, 0, :]
    x16 = jnp.pad(xf, ((0, NPAD - N), (0, COLS - P)))
    row = edge_index[0]
    col = edge_index[1]

    degp = _sc_degree(col)

    dinv, y16 = pl.pallas_call(
        _tc_prep_body,
        grid=(NPAD // BN,),
        in_specs=[
            pl.BlockSpec((NW, BN), lambda i: (0, i)),
            pl.BlockSpec((BN, COLS), lambda i: (i, 0)),
        ],
        out_specs=(
            pl.BlockSpec((BN, 1), lambda i: (i, 0)),
            pl.BlockSpec((BN, COLS), lambda i: (i, 0)),
        ),
        out_shape=(
            jax.ShapeDtypeStruct((NPAD, 1), jnp.float32),
            jax.ShapeDtypeStruct((NPAD, COLS), jnp.float32),
        ),
    )(degp, x16)

    tp = _sc_spmm(row, col, y16)

    probs = jax.nn.softmax(att)
    uz = (Wz @ Wlz[:OUT])[0]
    cz = bz @ Wlz[:OUT] + blz
    uh = (Wh @ Wlh[:OUT])[0]
    ch = bh @ Wlh[:OUT] + blh
    pidx = jnp.arange(P)
    col_ids = (pidx[:, None] * OUT + jnp.arange(OUT)[None, :]).reshape(-1)
    row_ids = jnp.repeat(pidx, OUT)
    EXP = jnp.zeros((COLS, 128), jnp.float32).at[row_ids, col_ids].set(1.0)
    UZ = jnp.zeros((1, 128), jnp.float32).at[0, col_ids].set(jnp.tile(uz, P))
    CZ = jnp.zeros((1, 128), jnp.float32).at[0, col_ids].set(jnp.tile(cz, P))
    UH = jnp.zeros((1, 128), jnp.float32).at[0, col_ids].set(jnp.tile(uh, P))
    CH2 = jnp.zeros((1, 128), jnp.float32).at[0, col_ids].set(jnp.tile(ch, P))
    wbv = (probs[:, None, None] * Wout[None, :, :]).reshape(P * OUT, P)
    WB = jnp.zeros((128, COLS), jnp.float32).at[:P * OUT, :P].set(wbv)
    BO = jnp.zeros((1, COLS), jnp.float32).at[0, :P].set(bout)

    out16 = pl.pallas_call(
        _tc_final_body,
        grid=(NPAD // BN,),
        in_specs=[
            pl.BlockSpec((NC, BN, COLS), lambda i: (0, i, 0)),
            pl.BlockSpec((BN, COLS), lambda i: (i, 0)),
            pl.BlockSpec((BN, 1), lambda i: (i, 0)),
            pl.BlockSpec((COLS, 128), lambda i: (0, 0)),
            pl.BlockSpec((1, 128), lambda i: (0, 0)),
            pl.BlockSpec((1, 128), lambda i: (0, 0)),
            pl.BlockSpec((1, 128), lambda i: (0, 0)),
            pl.BlockSpec((1, 128), lambda i: (0, 0)),
            pl.BlockSpec((128, COLS), lambda i: (0, 0)),
            pl.BlockSpec((1, COLS), lambda i: (0, 0)),
        ],
        out_specs=pl.BlockSpec((BN, COLS), lambda i: (i, 0)),
        out_shape=jax.ShapeDtypeStruct((NPAD, COLS), jnp.float32),
    )(tp, y16, dinv, EXP, UZ, CZ, UH, CH2, WB, BO)

    return out16[:N, :F_OUT]


# speedup vs baseline: 906.2970x; 1.3415x over previous
import functools

import jax
import jax.numpy as jnp
from jax import lax
from jax.experimental import pallas as pl
from jax.experimental.pallas import tpu as pltpu
from jax.experimental.pallas import tpu_sc as plsc

N = 50000
E = 1600000
P = 12
OUT = 4
F_OUT = 10

LANES = 16
NC = 2
NS = 16
NW = NC * NS
COLS = 16
NPAD = 51200
EPW = E // NW
CH = 2000
NCH = EPW // CH
RPT = NPAD // NS
BN = 3200

_mesh = plsc.VectorSubcoreMesh(core_axis_name="c", subcore_axis_name="s",
                               num_cores=NC, num_subcores=NS)
_sc_params = pltpu.CompilerParams(needs_layout_passes=False,
                                  use_tc_tiling_on_sc=False)


@functools.partial(
    pl.kernel,
    out_type=jax.ShapeDtypeStruct((NW, NPAD), jnp.float32),
    mesh=_mesh,
    scratch_types=[
        pltpu.VMEM((EPW,), jnp.int32),
        pltpu.VMEM((NPAD,), jnp.float32),
        pltpu.SemaphoreType.DMA,
    ],
    compiler_params=_sc_params,
)
def _sc_degree(ei_hbm, degp_hbm, colbuf, deg, sem):
    cid = lax.axis_index("c")
    sid = lax.axis_index("s")
    wid = sid * NC + cid
    zero = jnp.zeros((LANES,), jnp.float32)

    idx_dma = pltpu.async_copy(
        ei_hbm.at[1, pl.ds(wid * EPW, EPW)], colbuf, sem)

    def zb(i, c):
        deg[pl.ds(i * LANES, LANES)] = zero
        return c

    lax.fori_loop(0, NPAD // LANES, zb, 0, unroll=8)
    idx_dma.wait()

    ones = jnp.ones((LANES,), jnp.float32)

    def inner(i, c):
        idx = colbuf[pl.ds(i * LANES, LANES)]
        plsc.addupdate_scatter(deg, [idx], ones)
        return c

    lax.fori_loop(0, EPW // LANES, inner, 0, unroll=8)
    pltpu.sync_copy(deg, degp_hbm.at[wid])


@functools.partial(
    pl.kernel,
    out_type=jax.ShapeDtypeStruct((NC, NPAD, COLS), jnp.float32),
    mesh=_mesh,
    scratch_types=[
        pltpu.VMEM((3, CH), jnp.int32),
        pltpu.VMEM((3, CH), jnp.int32),
        pltpu.VMEM((2, CH, COLS), jnp.float32),
        pltpu.VMEM_SHARED((NPAD, COLS), jnp.float32),
        pltpu.SemaphoreType.DMA((3,)),
        pltpu.SemaphoreType.DMA((2,)),
        pltpu.SemaphoreType.DMA((2,)),
    ],
    compiler_params=_sc_params,
)
def _sc_spmm(ei_hbm, y_hbm, tp_hbm, rowbuf, colbuf, rows, t_sh,
             isem, gsem, ssem):
    cid = lax.axis_index("c")
    sid = lax.axis_index("s")
    wid = sid * NC + cid
    base0 = wid * EPW
    zero = jnp.zeros((LANES,), jnp.float32)

    def idx_start(ci):
        b = ci % 3
        d1 = pltpu.async_copy(
            ei_hbm.at[0, pl.ds(base0 + ci * CH, CH)], rowbuf.at[b], isem.at[b])
        d2 = pltpu.async_copy(
            ei_hbm.at[1, pl.ds(base0 + ci * CH, CH)], colbuf.at[b], isem.at[b])
        return d1, d2

    def gather_start(ci):
        return pltpu.async_copy(y_hbm.at[rowbuf.at[ci % 3]],
                                rows.at[ci % 2], gsem.at[ci % 2])

    def scatter_start(ci):
        return pltpu.async_copy(rows.at[ci % 2], t_sh.at[colbuf.at[ci % 3]],
                                ssem.at[ci % 2], add=True)

    idx_dmas = {0: idx_start(0), 1: idx_start(1)}

    def zb(i, c):
        rows[0, i] = zero
        return c

    lax.fori_loop(0, CH, zb, 0, unroll=8)
    pltpu.sync_copy(rows.at[0, pl.ds(0, CH)], t_sh.at[pl.ds(sid * RPT, CH)])
    pltpu.sync_copy(rows.at[0, pl.ds(0, RPT - CH)],
                    t_sh.at[pl.ds(sid * RPT + CH, RPT - CH)])
    plsc.subcore_barrier()

    for d in idx_dmas.pop(0):
        d.wait()
    gathers = {0: gather_start(0)}
    scatters = {}
    for ci in range(NCH):
        nxt = ci + 1
        if nxt < NCH:
            if ci >= 1:
                scatters.pop(ci - 1).wait()
            for d in idx_dmas.pop(nxt):
                d.wait()
            gathers[nxt] = gather_start(nxt)
            if nxt + 1 < NCH:
                idx_dmas[nxt + 1] = idx_start(nxt + 1)
        gathers.pop(ci).wait()
        scatters[ci] = scatter_start(ci)
    for ci in sorted(scatters):
        scatters.pop(ci).wait()

    plsc.subcore_barrier()
    pltpu.sync_copy(t_sh.at[pl.ds(sid * RPT, RPT)],
                    tp_hbm.at[cid, pl.ds(sid * RPT, RPT)])


def _tc_prep_body(degp_ref, x_ref, dinv_ref, y_ref):
    ones = jnp.ones((NW, 1), jnp.float32)
    deg = lax.dot_general(degp_ref[...], ones, (((0,), (0,)), ((), ())),
                          precision=lax.Precision.HIGHEST,
                          preferred_element_type=jnp.float32)
    dinv = lax.rsqrt(deg + 1.0)
    dinv_ref[...] = dinv
    xb = jnp.reshape(x_ref[...], (BN, P))
    x16 = jnp.concatenate([xb, jnp.zeros((BN, COLS - P), jnp.float32)],
                          axis=1)
    y_ref[...] = dinv * x16


def _tc_final_body(tp_ref, y_ref, dinv_ref, exp_ref, uz_ref, cz_ref,
                   uh_ref, ch_ref, wb_ref, bo_ref, out_ref):
    t = tp_ref[0] + tp_ref[1]
    s16 = dinv_ref[...] * (t + y_ref[...])
    se = jnp.dot(s16, exp_ref[...], precision=lax.Precision.HIGHEST,
                 preferred_element_type=jnp.float32)
    z = 1.0 / (1.0 + jnp.exp(-(se * uz_ref[...] + cz_ref[...])))
    ht = jnp.tanh(se * uh_ref[...] + ch_ref[...])
    m = (1.0 - z) * ht
    out_ref[...] = (jnp.dot(m, wb_ref[...], precision=lax.Precision.HIGHEST,
                            preferred_element_type=jnp.float32)
                    + bo_ref[...])


def kernel(x, edge_index, att, Wz, bz, Wr, br, Wh, bh,
           Wlz, blz, Wlr, blr, Wlh, blh, Wout, bout):
    degp = _sc_degree(edge_index)

    dinv, y16 = pl.pallas_call(
        _tc_prep_body,
        grid=(NPAD // BN,),
        in_specs=[
            pl.BlockSpec((NW, BN), lambda i: (0, i)),
            pl.BlockSpec((BN, 1, P), lambda i: (i, 0, 0)),
        ],
        out_specs=(
            pl.BlockSpec((BN, 1), lambda i: (i, 0)),
            pl.BlockSpec((BN, COLS), lambda i: (i, 0)),
        ),
        out_shape=(
            jax.ShapeDtypeStruct((NPAD, 1), jnp.float32),
            jax.ShapeDtypeStruct((NPAD, COLS), jnp.float32),
        ),
    )(degp, x)

    tp = _sc_spmm(edge_index, y16)

    probs = jax.nn.softmax(att)
    uz = (Wz @ Wlz[:OUT])[0]
    cz = bz @ Wlz[:OUT] + blz
    uh = (Wh @ Wlh[:OUT])[0]
    ch = bh @ Wlh[:OUT] + blh
    pidx = jnp.arange(P)
    col_ids = (pidx[:, None] * OUT + jnp.arange(OUT)[None, :]).reshape(-1)
    row_ids = jnp.repeat(pidx, OUT)
    EXP = jnp.zeros((COLS, 128), jnp.float32).at[row_ids, col_ids].set(1.0)
    UZ = jnp.zeros((1, 128), jnp.float32).at[0, col_ids].set(jnp.tile(uz, P))
    CZ = jnp.zeros((1, 128), jnp.float32).at[0, col_ids].set(jnp.tile(cz, P))
    UH = jnp.zeros((1, 128), jnp.float32).at[0, col_ids].set(jnp.tile(uh, P))
    CH2 = jnp.zeros((1, 128), jnp.float32).at[0, col_ids].set(jnp.tile(ch, P))
    wbv = (probs[:, None, None] * Wout[None, :, :]).reshape(P * OUT, P)
    WB = jnp.zeros((128, F_OUT), jnp.float32).at[:P * OUT].set(wbv[:, :F_OUT])
    BO = jnp.zeros((1, F_OUT), jnp.float32).at[0].set(bout[:F_OUT])

    out10 = pl.pallas_call(
        _tc_final_body,
        grid=(NPAD // BN,),
        in_specs=[
            pl.BlockSpec((NC, BN, COLS), lambda i: (0, i, 0)),
            pl.BlockSpec((BN, COLS), lambda i: (i, 0)),
            pl.BlockSpec((BN, 1), lambda i: (i, 0)),
            pl.BlockSpec((COLS, 128), lambda i: (0, 0)),
            pl.BlockSpec((1, 128), lambda i: (0, 0)),
            pl.BlockSpec((1, 128), lambda i: (0, 0)),
            pl.BlockSpec((1, 128), lambda i: (0, 0)),
            pl.BlockSpec((1, 128), lambda i: (0, 0)),
            pl.BlockSpec((128, F_OUT), lambda i: (0, 0)),
            pl.BlockSpec((1, F_OUT), lambda i: (0, 0)),
        ],
        out_specs=pl.BlockSpec((BN, F_OUT), lambda i: (i, 0)),
        out_shape=jax.ShapeDtypeStruct((NPAD, F_OUT), jnp.float32),
    )(tp, y16, dinv, EXP, UZ, CZ, UH, CH2, WB, BO)

    return out10[:N]


# speedup vs baseline: 1067.6076x; 1.1780x over previous
import functools

import jax
import jax.numpy as jnp
from jax import lax
from jax.experimental import pallas as pl
from jax.experimental.pallas import tpu as pltpu
from jax.experimental.pallas import tpu_sc as plsc

N = 50000
E = 1600000
P = 12
OUT = 4
F_OUT = 10

LANES = 16
NC = 2
NS = 16
NW = NC * NS
COLS = 16
NPAD = 51200
EPW = E // NW
CH = 2000
NCH = EPW // CH
RPT = NPAD // NS
BN = 3200

_mesh = plsc.VectorSubcoreMesh(core_axis_name="c", subcore_axis_name="s",
                               num_cores=NC, num_subcores=NS)
_sc_params = pltpu.CompilerParams(needs_layout_passes=False,
                                  use_tc_tiling_on_sc=False)


@functools.partial(
    pl.kernel,
    out_type=jax.ShapeDtypeStruct((NW, NPAD), jnp.float32),
    mesh=_mesh,
    scratch_types=[
        pltpu.VMEM((EPW,), jnp.int32),
        pltpu.VMEM((NPAD,), jnp.float32),
        pltpu.SemaphoreType.DMA,
    ],
    compiler_params=_sc_params,
)
def _sc_degree(ei_hbm, degp_hbm, colbuf, deg, sem):
    cid = lax.axis_index("c")
    sid = lax.axis_index("s")
    wid = sid * NC + cid
    zero = jnp.zeros((LANES,), jnp.float32)

    idx_dma = pltpu.async_copy(
        ei_hbm.at[1, pl.ds(wid * EPW, EPW)], colbuf, sem)

    def zb(i, c):
        deg[pl.ds(i * LANES, LANES)] = zero
        return c

    lax.fori_loop(0, NPAD // LANES, zb, 0, unroll=8)
    idx_dma.wait()

    ones = jnp.ones((LANES,), jnp.float32)

    def inner(i, c):
        idx = colbuf[pl.ds(i * LANES, LANES)]
        plsc.addupdate_scatter(deg, [idx], ones)
        return c

    lax.fori_loop(0, EPW // LANES, inner, 0, unroll=8)
    pltpu.sync_copy(deg, degp_hbm.at[wid])


@functools.partial(
    pl.kernel,
    out_type=jax.ShapeDtypeStruct((NC, NPAD, COLS), jnp.float32),
    mesh=_mesh,
    scratch_types=[
        pltpu.VMEM((3, CH), jnp.int32),
        pltpu.VMEM((3, CH), jnp.int32),
        pltpu.VMEM((2, CH, COLS), jnp.float32),
        pltpu.VMEM_SHARED((NPAD, COLS), jnp.float32),
        pltpu.SemaphoreType.DMA((3,)),
        pltpu.SemaphoreType.DMA((2,)),
        pltpu.SemaphoreType.DMA((2,)),
    ],
    compiler_params=_sc_params,
)
def _sc_spmm(ei_hbm, y_hbm, tp_hbm, rowbuf, colbuf, rows, t_sh,
             isem, gsem, ssem):
    cid = lax.axis_index("c")
    sid = lax.axis_index("s")
    wid = sid * NC + cid
    base0 = wid * EPW
    zero = jnp.zeros((LANES,), jnp.float32)

    def idx_start(ci):
        b = ci % 3
        d1 = pltpu.async_copy(
            ei_hbm.at[0, pl.ds(base0 + ci * CH, CH)], rowbuf.at[b], isem.at[b])
        d2 = pltpu.async_copy(
            ei_hbm.at[1, pl.ds(base0 + ci * CH, CH)], colbuf.at[b], isem.at[b])
        return d1, d2

    def gather_start(ci):
        return pltpu.async_copy(y_hbm.at[rowbuf.at[ci % 3]],
                                rows.at[ci % 2], gsem.at[ci % 2])

    def scatter_start(ci):
        return pltpu.async_copy(rows.at[ci % 2], t_sh.at[colbuf.at[ci % 3]],
                                ssem.at[ci % 2], add=True)

    idx_dmas = {0: idx_start(0), 1: idx_start(1)}

    def zb(i, c):
        rows[0, i] = zero
        return c

    lax.fori_loop(0, CH, zb, 0, unroll=8)
    pltpu.sync_copy(rows.at[0, pl.ds(0, CH)], t_sh.at[pl.ds(sid * RPT, CH)])
    pltpu.sync_copy(rows.at[0, pl.ds(0, RPT - CH)],
                    t_sh.at[pl.ds(sid * RPT + CH, RPT - CH)])
    plsc.subcore_barrier()

    for d in idx_dmas.pop(0):
        d.wait()
    gathers = {0: gather_start(0)}
    scatters = {}
    for ci in range(NCH):
        nxt = ci + 1
        if nxt < NCH:
            if ci >= 1:
                scatters.pop(ci - 1).wait()
            for d in idx_dmas.pop(nxt):
                d.wait()
            gathers[nxt] = gather_start(nxt)
            if nxt + 1 < NCH:
                idx_dmas[nxt + 1] = idx_start(nxt + 1)
        gathers.pop(ci).wait()
        scatters[ci] = scatter_start(ci)
    for ci in sorted(scatters):
        scatters.pop(ci).wait()

    plsc.subcore_barrier()
    pltpu.sync_copy(t_sh.at[pl.ds(sid * RPT, RPT)],
                    tp_hbm.at[cid, pl.ds(sid * RPT, RPT)])


WB_ROWS = BN // 8
WD = NPAD // 128


def _tc_prep_body(degp_ref, x_ref, dinv16_ref, y_ref):
    ones = jnp.ones((NW, 1), jnp.float32)
    deg = lax.dot_general(degp_ref[...], ones, (((0,), (0,)), ((), ())),
                          precision=lax.Precision.HIGHEST,
                          preferred_element_type=jnp.float32)
    dinv = lax.rsqrt(deg + 1.0)
    dinv16_ref[...] = jnp.broadcast_to(dinv, (BN, COLS))
    xb = jnp.reshape(x_ref[...], (BN, P))
    x16 = jnp.concatenate([xb, jnp.zeros((BN, COLS - P), jnp.float32)],
                          axis=1)
    y_ref[...] = dinv * x16


def _tc_final_body(tp_ref, yw_ref, dinvw_ref, expw_ref, uz_ref, cz_ref,
                   uh_ref, ch_ref, wb_ref, bo_ref, out_ref):
    t = tp_ref[0] + tp_ref[1]
    sw = dinvw_ref[...] * (t + yw_ref[...])
    sew = jnp.dot(sw, expw_ref[...], precision=lax.Precision.HIGHEST,
                  preferred_element_type=jnp.float32)
    z = 1.0 / (1.0 + jnp.exp(-(sew * uz_ref[...] + cz_ref[...])))
    ht = jnp.tanh(sew * uh_ref[...] + ch_ref[...])
    m2 = jnp.reshape((1.0 - z) * ht, (BN, 128))
    out_ref[...] = (jnp.dot(m2, wb_ref[...], precision=lax.Precision.HIGHEST,
                            preferred_element_type=jnp.float32)
                    + bo_ref[...])


def kernel(x, edge_index, att, Wz, bz, Wr, br, Wh, bh,
           Wlz, blz, Wlr, blr, Wlh, blh, Wout, bout):
    degp = _sc_degree(edge_index)

    dinv16, y16 = pl.pallas_call(
        _tc_prep_body,
        grid=(NPAD // BN,),
        in_specs=[
            pl.BlockSpec((NW, BN), lambda i: (0, i)),
            pl.BlockSpec((BN, 1, P), lambda i: (i, 0, 0)),
        ],
        out_specs=(
            pl.BlockSpec((BN, COLS), lambda i: (i, 0)),
            pl.BlockSpec((BN, COLS), lambda i: (i, 0)),
        ),
        out_shape=(
            jax.ShapeDtypeStruct((NPAD, COLS), jnp.float32),
            jax.ShapeDtypeStruct((NPAD, COLS), jnp.float32),
        ),
    )(degp, x)

    tp = _sc_spmm(edge_index, y16)
    tpw = tp.reshape(NC, NPAD // 8, 128)
    yw = y16.reshape(NPAD // 8, 128)
    dinvw = dinv16.reshape(NPAD // 8, 128)

    probs = jax.nn.softmax(att)
    uz = (Wz @ Wlz[:OUT])[0]
    cz = bz @ Wlz[:OUT] + blz
    uh = (Wh @ Wlh[:OUT])[0]
    ch = bh @ Wlh[:OUT] + blh
    pidx = jnp.arange(P)
    col_ids = (pidx[:, None] * OUT + jnp.arange(OUT)[None, :]).reshape(-1)
    row_ids = jnp.repeat(pidx, OUT)
    EXP = jnp.zeros((COLS, 128), jnp.float32).at[row_ids, col_ids].set(1.0)
    UZ = jnp.zeros((1, 128), jnp.float32).at[0, col_ids].set(jnp.tile(uz, P))
    CZ = jnp.zeros((1, 128), jnp.float32).at[0, col_ids].set(jnp.tile(cz, P))
    UH = jnp.zeros((1, 128), jnp.float32).at[0, col_ids].set(jnp.tile(uh, P))
    CH2 = jnp.zeros((1, 128), jnp.float32).at[0, col_ids].set(jnp.tile(ch, P))
    EXPW = jnp.kron(jnp.eye(8, dtype=jnp.float32), EXP)
    UZW = jnp.tile(UZ, (1, 8))
    CZW = jnp.tile(CZ, (1, 8))
    UHW = jnp.tile(UH, (1, 8))
    CHW = jnp.tile(CH2, (1, 8))
    wbv = (probs[:, None, None] * Wout[None, :, :]).reshape(P * OUT, P)
    WB = jnp.zeros((128, F_OUT), jnp.float32).at[:P * OUT].set(wbv[:, :F_OUT])
    BO = jnp.zeros((1, F_OUT), jnp.float32).at[0].set(bout[:F_OUT])

    out10 = pl.pallas_call(
        _tc_final_body,
        grid=(NPAD // BN,),
        in_specs=[
            pl.BlockSpec((NC, WB_ROWS, 128), lambda i: (0, i, 0)),
            pl.BlockSpec((WB_ROWS, 128), lambda i: (i, 0)),
            pl.BlockSpec((WB_ROWS, 128), lambda i: (i, 0)),
            pl.BlockSpec((128, 1024), lambda i: (0, 0)),
            pl.BlockSpec((1, 1024), lambda i: (0, 0)),
            pl.BlockSpec((1, 1024), lambda i: (0, 0)),
            pl.BlockSpec((1, 1024), lambda i: (0, 0)),
            pl.BlockSpec((1, 1024), lambda i: (0, 0)),
            pl.BlockSpec((128, F_OUT), lambda i: (0, 0)),
            pl.BlockSpec((1, F_OUT), lambda i: (0, 0)),
        ],
        out_specs=pl.BlockSpec((BN, F_OUT), lambda i: (i, 0)),
        out_shape=jax.ShapeDtypeStruct((NPAD, F_OUT), jnp.float32),
    )(tpw, yw, dinvw, EXPW, UZW, CZW, UHW, CHW, WB, BO)

    return out10[:N]


# speedup vs baseline: 1125.7032x; 1.0544x over previous
import functools

import jax
import jax.numpy as jnp
from jax import lax
from jax.experimental import pallas as pl
from jax.experimental.pallas import tpu as pltpu
from jax.experimental.pallas import tpu_sc as plsc

N = 50000
E = 1600000
P = 12
OUT = 4
F_OUT = 10

LANES = 16
NC = 2
NS = 16
NW = NC * NS
COLS = 16
NPAD = 51200
EPW = E // NW
CH = 2000
NCH = EPW // CH
RPT = NPAD // NS
BN = 3200

_mesh = plsc.VectorSubcoreMesh(core_axis_name="c", subcore_axis_name="s",
                               num_cores=NC, num_subcores=NS)
_sc_params = pltpu.CompilerParams(needs_layout_passes=False,
                                  use_tc_tiling_on_sc=False)


@functools.partial(
    pl.kernel,
    out_type=jax.ShapeDtypeStruct((NW, NPAD), jnp.float32),
    mesh=_mesh,
    scratch_types=[
        pltpu.VMEM((EPW,), jnp.int32),
        pltpu.VMEM((NPAD,), jnp.float32),
        pltpu.SemaphoreType.DMA,
    ],
    compiler_params=_sc_params,
)
def _sc_degree(ei_hbm, degp_hbm, colbuf, deg, sem):
    cid = lax.axis_index("c")
    sid = lax.axis_index("s")
    wid = sid * NC + cid
    zero = jnp.zeros((LANES,), jnp.float32)

    idx_dma = pltpu.async_copy(
        ei_hbm.at[1, pl.ds(wid * EPW, EPW)], colbuf, sem)

    def zb(i, c):
        deg[pl.ds(i * LANES, LANES)] = zero
        return c

    lax.fori_loop(0, NPAD // LANES, zb, 0, unroll=8)
    idx_dma.wait()

    ones = jnp.ones((LANES,), jnp.float32)

    def inner(i, c):
        idx = colbuf[pl.ds(i * LANES, LANES)]
        plsc.addupdate_scatter(deg, [idx], ones)
        return c

    lax.fori_loop(0, EPW // LANES, inner, 0, unroll=8)
    pltpu.sync_copy(deg, degp_hbm.at[wid])


@functools.partial(
    pl.kernel,
    out_type=jax.ShapeDtypeStruct((NC, NPAD, COLS), jnp.float32),
    mesh=_mesh,
    scratch_types=[
        pltpu.VMEM((3, CH), jnp.int32),
        pltpu.VMEM((3, CH), jnp.int32),
        pltpu.VMEM((2, CH, COLS), jnp.float32),
        pltpu.VMEM_SHARED((NPAD, COLS), jnp.float32),
        pltpu.SemaphoreType.DMA((3,)),
        pltpu.SemaphoreType.DMA((2,)),
        pltpu.SemaphoreType.DMA((2,)),
    ],
    compiler_params=_sc_params,
)
def _sc_spmm(ei_hbm, y_hbm, tp_hbm, rowbuf, colbuf, rows, t_sh,
             isem, gsem, ssem):
    cid = lax.axis_index("c")
    sid = lax.axis_index("s")
    wid = sid * NC + cid
    base0 = wid * EPW
    zero = jnp.zeros((LANES,), jnp.float32)

    def idx_start(ci):
        b = ci % 3
        d1 = pltpu.async_copy(
            ei_hbm.at[0, pl.ds(base0 + ci * CH, CH)], rowbuf.at[b], isem.at[b])
        d2 = pltpu.async_copy(
            ei_hbm.at[1, pl.ds(base0 + ci * CH, CH)], colbuf.at[b], isem.at[b])
        return d1, d2

    def gather_start(ci):
        return pltpu.async_copy(y_hbm.at[rowbuf.at[ci % 3]],
                                rows.at[ci % 2], gsem.at[ci % 2])

    def scatter_start(ci):
        return pltpu.async_copy(rows.at[ci % 2], t_sh.at[colbuf.at[ci % 3]],
                                ssem.at[ci % 2], add=True)

    idx_dmas = {0: idx_start(0), 1: idx_start(1)}

    def zb(i, c):
        rows[0, i] = zero
        return c

    lax.fori_loop(0, CH, zb, 0, unroll=8)
    pltpu.sync_copy(rows.at[0, pl.ds(0, CH)], t_sh.at[pl.ds(sid * RPT, CH)])
    pltpu.sync_copy(rows.at[0, pl.ds(0, RPT - CH)],
                    t_sh.at[pl.ds(sid * RPT + CH, RPT - CH)])
    plsc.subcore_barrier()

    for d in idx_dmas.pop(0):
        d.wait()
    gathers = {0: gather_start(0)}
    scatters = {}
    for ci in range(NCH):
        nxt = ci + 1
        if nxt < NCH:
            if ci >= 1:
                scatters.pop(ci - 1).wait()
            for d in idx_dmas.pop(nxt):
                d.wait()
            gathers[nxt] = gather_start(nxt)
            if nxt + 1 < NCH:
                idx_dmas[nxt + 1] = idx_start(nxt + 1)
        gathers.pop(ci).wait()
        scatters[ci] = scatter_start(ci)
    for ci in sorted(scatters):
        scatters.pop(ci).wait()

    plsc.subcore_barrier()
    pltpu.sync_copy(t_sh.at[pl.ds(sid * RPT, RPT)],
                    tp_hbm.at[cid, pl.ds(sid * RPT, RPT)])


WB_ROWS = BN // 8
WD = NPAD // 128


def _tc_prep_body(degp_ref, x_ref, dinv16_ref, y_ref):
    ones = jnp.ones((NW, 1), jnp.float32)
    deg = lax.dot_general(degp_ref[...], ones, (((0,), (0,)), ((), ())),
                          precision=lax.Precision.HIGHEST,
                          preferred_element_type=jnp.float32)
    dinv = lax.rsqrt(deg + 1.0)
    dinv16_ref[...] = jnp.broadcast_to(dinv, (BN, COLS))
    xb = jnp.reshape(x_ref[...], (BN, P))
    x16 = jnp.concatenate([xb, jnp.zeros((BN, COLS - P), jnp.float32)],
                          axis=1)
    y_ref[...] = dinv * x16


def _tc_final_body(tp_ref, yw_ref, dinvw_ref, expw_ref, uz_ref, cz_ref,
                   uh_ref, ch_ref, wb_ref, bo_ref, out_ref):
    t = tp_ref[0] + tp_ref[1]
    sw = dinvw_ref[...] * (t + yw_ref[...])
    sew = jnp.dot(sw, expw_ref[...], precision=lax.Precision.HIGHEST,
                  preferred_element_type=jnp.float32)
    z = 1.0 / (1.0 + jnp.exp(-(sew * uz_ref[...] + cz_ref[...])))
    ht = jnp.tanh(sew * uh_ref[...] + ch_ref[...])
    m2 = jnp.reshape((1.0 - z) * ht, (BN, 128))
    out_ref[...] = (jnp.dot(m2, wb_ref[...], precision=lax.Precision.HIGHEST,
                            preferred_element_type=jnp.float32)
                    + bo_ref[...])


def kernel(x, edge_index, att, Wz, bz, Wr, br, Wh, bh,
           Wlz, blz, Wlr, blr, Wlh, blh, Wout, bout):
    degp = _sc_degree(edge_index)

    dinv16, y16 = pl.pallas_call(
        _tc_prep_body,
        grid=(NPAD // BN,),
        in_specs=[
            pl.BlockSpec((NW, BN), lambda i: (0, i)),
            pl.BlockSpec((BN, 1, P), lambda i: (i, 0, 0)),
        ],
        out_specs=(
            pl.BlockSpec((BN, COLS), lambda i: (i, 0)),
            pl.BlockSpec((BN, COLS), lambda i: (i, 0)),
        ),
        out_shape=(
            jax.ShapeDtypeStruct((NPAD, COLS), jnp.float32),
            jax.ShapeDtypeStruct((NPAD, COLS), jnp.float32),
        ),
    )(degp, x)

    tp = _sc_spmm(edge_index, y16)
    tpw = tp.reshape(NC, NPAD // 8, 128)
    yw = y16.reshape(NPAD // 8, 128)
    dinvw = dinv16.reshape(NPAD // 8, 128)

    probs = jax.nn.softmax(att)
    uz = (Wz @ Wlz[:OUT])[0]
    cz = bz @ Wlz[:OUT] + blz
    uh = (Wh @ Wlh[:OUT])[0]
    ch = bh @ Wlh[:OUT] + blh
    pidx = jnp.arange(P)
    col_ids = (pidx[:, None] * OUT + jnp.arange(OUT)[None, :]).reshape(-1)
    row_ids = jnp.repeat(pidx, OUT)
    EXP = jnp.zeros((COLS, 128), jnp.float32).at[row_ids, col_ids].set(1.0)
    UZ = jnp.zeros((1, 128), jnp.float32).at[0, col_ids].set(jnp.tile(uz, P))
    CZ = jnp.zeros((1, 128), jnp.float32).at[0, col_ids].set(jnp.tile(cz, P))
    UH = jnp.zeros((1, 128), jnp.float32).at[0, col_ids].set(jnp.tile(uh, P))
    CH2 = jnp.zeros((1, 128), jnp.float32).at[0, col_ids].set(jnp.tile(ch, P))
    EXPW = jnp.kron(jnp.eye(8, dtype=jnp.float32), EXP)
    UZW = jnp.tile(UZ, (1, 8))
    CZW = jnp.tile(CZ, (1, 8))
    UHW = jnp.tile(UH, (1, 8))
    CHW = jnp.tile(CH2, (1, 8))
    wbv = (probs[:, None, None] * Wout[None, :, :]).reshape(P * OUT, P)
    WB = jnp.zeros((128, F_OUT), jnp.float32).at[:P * OUT].set(wbv[:, :F_OUT])
    BO = jnp.zeros((1, F_OUT), jnp.float32).at[0].set(bout[:F_OUT])

    out10 = pl.pallas_call(
        _tc_final_body,
        grid=(NPAD // BN,),
        in_specs=[
            pl.BlockSpec((NC, WB_ROWS, 128), lambda i: (0, i, 0)),
            pl.BlockSpec((WB_ROWS, 128), lambda i: (i, 0)),
            pl.BlockSpec((WB_ROWS, 128), lambda i: (i, 0)),
            pl.BlockSpec((128, 1024), lambda i: (0, 0)),
            pl.BlockSpec((1, 1024), lambda i: (0, 0)),
            pl.BlockSpec((1, 1024), lambda i: (0, 0)),
            pl.BlockSpec((1, 1024), lambda i: (0, 0)),
            pl.BlockSpec((1, 1024), lambda i: (0, 0)),
            pl.BlockSpec((128, F_OUT), lambda i: (0, 0)),
            pl.BlockSpec((1, F_OUT), lambda i: (0, 0)),
        ],
        out_specs=pl.BlockSpec((BN, F_OUT), lambda i: (i, 0)),
        out_shape=jax.ShapeDtypeStruct((N, F_OUT), jnp.float32),
    )(tpw, yw, dinvw, EXPW, UZW, CZW, UHW, CHW, WB, BO)

    return out10
